# baseline (speedup 1.0000x reference)
import functools

import jax
import jax.numpy as jnp
from jax import lax
from jax.experimental import pallas as pl
from jax.experimental.pallas import tpu as pltpu
from jax.experimental.pallas import tpu_sc as plsc

N = 10000
D = 128
E = 320000

NC = 2
NS = 16
NW = NC * NS
EPW = E // NW
CHUNK = 128
EPW_PAD = 10240
NCHUNK = EPW_PAD // CHUNK
NPAD = 10240
RPS = NPAD // NS

@functools.cache
def _make_sc_aggregate():
    mesh = plsc.VectorSubcoreMesh(
        core_axis_name="c", subcore_axis_name="s",
        num_cores=NC, num_subcores=NS,
    )
    return pl.kernel(
        _sc_aggregate_body,
        out_type=jax.ShapeDtypeStruct((NC, NPAD, D), jnp.float32),
        mesh=mesh,
        scratch_types=[
            pltpu.VMEM((4, 2, CHUNK), jnp.int32),
            pltpu.VMEM((CHUNK, D), jnp.float32),
            pltpu.VMEM((CHUNK, D), jnp.float32),
            pltpu.VMEM_SHARED((NPAD, D), jnp.float32),
            [pltpu.SemaphoreType.DMA] * 4,
            pltpu.SemaphoreType.DMA,
            pltpu.SemaphoreType.DMA,
        ],
    )


def _sc_aggregate_body(table_hbm, eidx_hbm, zeros_hbm, out_hbm,
                       ring, buf_a, buf_b, agg_sh, isems, gsem_a, gsem_b):
    c = lax.axis_index("c")
    s = lax.axis_index("s")
    wid = s * NC + c

    def idx_fetch(chunk, slot):
        pltpu.async_copy(eidx_hbm.at[wid, jnp.minimum(chunk, NCHUNK - 1)],
                         ring.at[slot], isems[slot])

    def wait_idx(slot):
        pltpu.make_async_copy(eidx_hbm.at[wid, 0], ring.at[slot],
                              isems[slot]).wait()

    def gather(slot, buf, gsem):
        pltpu.async_copy(table_hbm.at[ring.at[slot, 0]], buf, gsem)

    def wait_gather(buf, gsem):
        pltpu.make_async_copy(table_hbm.at[ring.at[0, 0]], buf, gsem).wait()

    def scatter(buf, slot):
        pltpu.sync_copy(buf, agg_sh.at[ring.at[slot, 1]], add=True)

    pltpu.sync_copy(zeros_hbm, agg_sh.at[pl.ds(s * RPS, RPS)])

    for p in range(4):
        idx_fetch(p, p)
    wait_idx(0)
    gather(0, buf_a, gsem_a)
    wait_idx(1)
    gather(1, buf_b, gsem_b)
    plsc.subcore_barrier()

    def body(k, _):
        j = 4 * k
        wait_gather(buf_a, gsem_a)
        scatter(buf_a, 0)
        wait_idx(2)
        gather(2, buf_a, gsem_a)
        idx_fetch(j + 4, 0)

        wait_gather(buf_b, gsem_b)
        scatter(buf_b, 1)
        wait_idx(3)
        gather(3, buf_b, gsem_b)
        idx_fetch(j + 5, 1)

        wait_gather(buf_a, gsem_a)
        scatter(buf_a, 2)
        wait_idx(0)
        gather(0, buf_a, gsem_a)
        idx_fetch(j + 6, 2)

        wait_gather(buf_b, gsem_b)
        scatter(buf_b, 3)
        wait_idx(1)
        gather(1, buf_b, gsem_b)
        idx_fetch(j + 7, 3)
        return 0

    lax.fori_loop(0, NCHUNK // 4, body, 0)

    wait_gather(buf_a, gsem_a)
    wait_gather(buf_b, gsem_b)
    wait_idx(2)
    wait_idx(3)

    plsc.subcore_barrier()
    pltpu.sync_copy(agg_sh.at[pl.ds(s * RPS, RPS)],
                    out_hbm.at[c, pl.ds(s * RPS, RPS)])


def _mm_relu_body(x_ref, a_ref, w_ref, b_ref, o_ref):
    xa = x_ref[...] + a_ref[0] + a_ref[1]
    h = jnp.dot(xa, w_ref[...], preferred_element_type=jnp.float32)
    o_ref[...] = jnp.maximum(h + b_ref[...], 0.0)


def _mm_lsm_body(x_ref, a_ref, w_ref, b_ref, o_ref):
    xa = x_ref[...] + a_ref[0] + a_ref[1]
    z = jnp.dot(xa, w_ref[...], preferred_element_type=jnp.float32)
    z = z + b_ref[...]
    m = jnp.max(z, axis=1, keepdims=True)
    lse = jnp.log(jnp.sum(jnp.exp(z - m), axis=1, keepdims=True)) + m
    o_ref[...] = z - lse


ROWS_BLK = 1000


def _tc_layer(body, x, aggp, wt, b):
    return pl.pallas_call(
        body,
        out_shape=jax.ShapeDtypeStruct((N, D), jnp.float32),
        grid=(N // ROWS_BLK,),
        in_specs=[
            pl.BlockSpec((ROWS_BLK, D), lambda i: (i, 0)),
            pl.BlockSpec((NC, ROWS_BLK, D), lambda i: (0, i, 0)),
            pl.BlockSpec((D, D), lambda i: (0, 0)),
            pl.BlockSpec((1, D), lambda i: (0, 0)),
        ],
        out_specs=pl.BlockSpec((ROWS_BLK, D), lambda i: (i, 0)),
    )(x, aggp, wt, b)


def kernel(x, edge_index, W1, b1, W2, b2):
    ei = edge_index.astype(jnp.int32)
    pad = EPW_PAD - EPW
    src = jnp.pad(ei[0].reshape(NW, EPW), ((0, 0), (0, pad)),
                  constant_values=0).reshape(NW, NCHUNK, CHUNK)
    pad_dst = (N + jnp.arange(pad, dtype=jnp.int32) % (NPAD - N))[None, :]
    dst = jnp.concatenate(
        [ei[1].reshape(NW, EPW), jnp.broadcast_to(pad_dst, (NW, pad))],
        axis=1).reshape(NW, NCHUNK, CHUNK)
    eidx = jnp.stack([src, dst], axis=2)
    zeros = jnp.zeros((RPS, D), dtype=jnp.float32)

    sc_aggregate = _make_sc_aggregate()
    agg1 = sc_aggregate(x, eidx, zeros)
    h = _tc_layer(_mm_relu_body, x, agg1, W1.T, b1.reshape(1, D))
    agg2 = sc_aggregate(h, eidx, zeros)
    out = _tc_layer(_mm_lsm_body, h, agg2, W2.T, b2.reshape(1, D))
    return out


# speedup vs baseline: 5.0270x; 5.0270x over previous
import functools

import jax
import jax.numpy as jnp
from jax import lax
from jax.experimental import pallas as pl
from jax.experimental.pallas import tpu as pltpu
from jax.experimental.pallas import tpu_sc as plsc

N = 10000
D = 128
E = 320000

NC = 2
NS = 16
NW = NC * NS
EPW = E // NW
CHUNK = 100
EPW_PAD = EPW
NCHUNK = EPW_PAD // CHUNK
NPAD = 10240
RPS = NPAD // NS

@functools.cache
def _make_sc_aggregate():
    mesh = plsc.VectorSubcoreMesh(
        core_axis_name="c", subcore_axis_name="s",
        num_cores=NC, num_subcores=NS,
    )
    return pl.kernel(
        _sc_aggregate_body,
        out_type=jax.ShapeDtypeStruct((NC, NPAD, D), jnp.float32),
        mesh=mesh,
        scratch_types=[
            pltpu.VMEM((4, 2, CHUNK), jnp.int32),
            pltpu.VMEM((CHUNK, D), jnp.float32),
            pltpu.VMEM((CHUNK, D), jnp.float32),
            pltpu.VMEM_SHARED((NPAD, D), jnp.float32),
            [pltpu.SemaphoreType.DMA] * 4,
            pltpu.SemaphoreType.DMA,
            pltpu.SemaphoreType.DMA,
        ],
    )


def _sc_aggregate_body(table_hbm, eidx_hbm, zeros_hbm, out_hbm,
                       ring, buf_a, buf_b, agg_sh, isems, gsem_a, gsem_b):
    c = lax.axis_index("c")
    s = lax.axis_index("s")
    wid = s * NC + c

    def idx_fetch(chunk, slot):
        pltpu.async_copy(eidx_hbm.at[wid, jnp.minimum(chunk, NCHUNK - 1)],
                         ring.at[slot], isems[slot])

    def wait_idx(slot):
        pltpu.make_async_copy(eidx_hbm.at[wid, 0], ring.at[slot],
                              isems[slot]).wait()

    def gather(slot, buf, gsem):
        pltpu.async_copy(table_hbm.at[ring.at[slot, 0]], buf, gsem)

    def wait_gather(buf, gsem):
        pltpu.make_async_copy(table_hbm.at[ring.at[0, 0]], buf, gsem).wait()

    def scatter(buf, slot):
        pltpu.sync_copy(buf, agg_sh.at[ring.at[slot, 1]], add=True)

    pltpu.sync_copy(zeros_hbm, agg_sh.at[pl.ds(s * RPS, RPS)])

    for p in range(4):
        idx_fetch(p, p)
    wait_idx(0)
    gather(0, buf_a, gsem_a)
    wait_idx(1)
    gather(1, buf_b, gsem_b)
    plsc.subcore_barrier()

    def body(k, _):
        j = 4 * k
        wait_gather(buf_a, gsem_a)
        scatter(buf_a, 0)
        wait_idx(2)
        gather(2, buf_a, gsem_a)
        idx_fetch(j + 4, 0)

        wait_gather(buf_b, gsem_b)
        scatter(buf_b, 1)
        wait_idx(3)
        gather(3, buf_b, gsem_b)
        idx_fetch(j + 5, 1)

        wait_gather(buf_a, gsem_a)
        scatter(buf_a, 2)
        wait_idx(0)
        gather(0, buf_a, gsem_a)
        idx_fetch(j + 6, 2)

        wait_gather(buf_b, gsem_b)
        scatter(buf_b, 3)
        wait_idx(1)
        gather(1, buf_b, gsem_b)
        idx_fetch(j + 7, 3)
        return 0

    lax.fori_loop(0, NCHUNK // 4, body, 0)

    wait_gather(buf_a, gsem_a)
    wait_gather(buf_b, gsem_b)
    wait_idx(2)
    wait_idx(3)

    plsc.subcore_barrier()
    pltpu.sync_copy(agg_sh.at[pl.ds(s * RPS, RPS)],
                    out_hbm.at[c, pl.ds(s * RPS, RPS)])


def _mm_relu_body(x_ref, a_ref, w_ref, b_ref, o_ref):
    xa = x_ref[...] + a_ref[0] + a_ref[1]
    h = jnp.dot(xa, w_ref[...], preferred_element_type=jnp.float32)
    o_ref[...] = jnp.maximum(h + b_ref[...], 0.0)


def _mm_lsm_body(x_ref, a_ref, w_ref, b_ref, o_ref):
    xa = x_ref[...] + a_ref[0] + a_ref[1]
    z = jnp.dot(xa, w_ref[...], preferred_element_type=jnp.float32)
    z = z + b_ref[...]
    m = jnp.max(z, axis=1, keepdims=True)
    lse = jnp.log(jnp.sum(jnp.exp(z - m), axis=1, keepdims=True)) + m
    o_ref[...] = z - lse


ROWS_BLK = 1000


def _tc_layer(body, x, aggp, wt, b):
    return pl.pallas_call(
        body,
        out_shape=jax.ShapeDtypeStruct((N, D), jnp.float32),
        grid=(N // ROWS_BLK,),
        in_specs=[
            pl.BlockSpec((ROWS_BLK, D), lambda i: (i, 0)),
            pl.BlockSpec((NC, ROWS_BLK, D), lambda i: (0, i, 0)),
            pl.BlockSpec((D, D), lambda i: (0, 0)),
            pl.BlockSpec((1, D), lambda i: (0, 0)),
        ],
        out_specs=pl.BlockSpec((ROWS_BLK, D), lambda i: (i, 0)),
    )(x, aggp, wt, b)


def kernel(x, edge_index, W1, b1, W2, b2):
    ei = edge_index.astype(jnp.int32)
    src = ei[0].reshape(NW, NCHUNK, CHUNK)
    dst = ei[1].reshape(NW, NCHUNK, CHUNK)
    eidx = jnp.stack([src, dst], axis=2)
    zeros = jnp.zeros((RPS, D), dtype=jnp.float32)

    sc_aggregate = _make_sc_aggregate()
    agg1 = sc_aggregate(x, eidx, zeros)
    h = _tc_layer(_mm_relu_body, x, agg1, W1.T, b1.reshape(1, D))
    agg2 = sc_aggregate(h, eidx, zeros)
    out = _tc_layer(_mm_lsm_body, h, agg2, W2.T, b2.reshape(1, D))
    return out


# speedup vs baseline: 5.0591x; 1.0064x over previous
import functools

import jax
import jax.numpy as jnp
from jax import lax
from jax.experimental import pallas as pl
from jax.experimental.pallas import tpu as pltpu
from jax.experimental.pallas import tpu_sc as plsc

N = 10000
D = 128
E = 320000

NC = 2
NS = 16
NW = NC * NS
EPW = E // NW
CHUNK = 100
EPW_PAD = EPW
NCHUNK = EPW_PAD // CHUNK
NPAD = 10240
RPS = NPAD // NS

@functools.cache
def _make_sc_aggregate():
    mesh = plsc.VectorSubcoreMesh(
        core_axis_name="c", subcore_axis_name="s",
        num_cores=NC, num_subcores=NS,
    )
    return pl.kernel(
        _sc_aggregate_body,
        out_type=jax.ShapeDtypeStruct((NC, NPAD, D), jnp.float32),
        mesh=mesh,
        scratch_types=[
            pltpu.VMEM((2, 4, 2, CHUNK), jnp.int32),
            pltpu.VMEM((CHUNK, D), jnp.float32),
            pltpu.VMEM((CHUNK, D), jnp.float32),
            pltpu.VMEM_SHARED((NPAD, D), jnp.float32),
            [pltpu.SemaphoreType.DMA] * 2,
            pltpu.SemaphoreType.DMA,
            pltpu.SemaphoreType.DMA,
            pltpu.SemaphoreType.DMA,
        ],
    )


def _sc_aggregate_body(table_hbm, eidx_hbm, zeros_hbm, out_hbm,
                       ring, buf_a, buf_b, agg_sh, isems, gsem_a, gsem_b, zsem):
    c = lax.axis_index("c")
    s = lax.axis_index("s")
    wid = s * NC + c

    def group_fetch(group, half):
        g = jnp.minimum(group, NCHUNK // 4 - 1)
        pltpu.async_copy(eidx_hbm.at[wid, pl.ds(4 * g, 4)], ring.at[half],
                         isems[half])

    def wait_group(half):
        pltpu.make_async_copy(eidx_hbm.at[wid, pl.ds(0, 4)], ring.at[half],
                              isems[half]).wait()

    def gather(half, m, buf, gsem):
        pltpu.async_copy(table_hbm.at[ring.at[half, m, 0]], buf, gsem)

    def wait_gather(buf, gsem):
        pltpu.make_async_copy(table_hbm.at[ring.at[0, 0, 0]], buf, gsem).wait()

    def scatter(buf, half, m):
        pltpu.sync_copy(buf, agg_sh.at[ring.at[half, m, 1]], add=True)

    init_cp = pltpu.async_copy(zeros_hbm, agg_sh.at[pl.ds(s * RPS, RPS)], zsem)

    group_fetch(0, 0)
    group_fetch(1, 1)
    wait_group(0)
    gather(0, 0, buf_a, gsem_a)
    gather(0, 1, buf_b, gsem_b)
    init_cp.wait()
    plsc.subcore_barrier()

    def one_group(g, h):
        h2 = 1 - h
        wait_gather(buf_a, gsem_a)
        scatter(buf_a, h, 0)
        gather(h, 2, buf_a, gsem_a)

        wait_gather(buf_b, gsem_b)
        scatter(buf_b, h, 1)
        gather(h, 3, buf_b, gsem_b)

        wait_gather(buf_a, gsem_a)
        scatter(buf_a, h, 2)
        wait_group(h2)
        gather(h2, 0, buf_a, gsem_a)

        wait_gather(buf_b, gsem_b)
        scatter(buf_b, h, 3)
        gather(h2, 1, buf_b, gsem_b)

        group_fetch(g + 2, h)

    def body(m, _):
        one_group(2 * m, 0)
        one_group(2 * m + 1, 1)
        return 0

    lax.fori_loop(0, NCHUNK // 8, body, 0)
    one_group(NCHUNK // 4 - 1, 0)

    wait_gather(buf_a, gsem_a)
    wait_gather(buf_b, gsem_b)
    wait_group(0)

    plsc.subcore_barrier()
    pltpu.sync_copy(agg_sh.at[pl.ds(s * RPS, RPS)],
                    out_hbm.at[c, pl.ds(s * RPS, RPS)])


def _mm_relu_body(x_ref, a_ref, w_ref, b_ref, o_ref):
    xa = x_ref[...] + a_ref[0] + a_ref[1]
    h = jnp.dot(xa, w_ref[...], preferred_element_type=jnp.float32)
    o_ref[...] = jnp.maximum(h + b_ref[...], 0.0)


def _mm_lsm_body(x_ref, a_ref, w_ref, b_ref, o_ref):
    xa = x_ref[...] + a_ref[0] + a_ref[1]
    z = jnp.dot(xa, w_ref[...], preferred_element_type=jnp.float32)
    z = z + b_ref[...]
    m = jnp.max(z, axis=1, keepdims=True)
    lse = jnp.log(jnp.sum(jnp.exp(z - m), axis=1, keepdims=True)) + m
    o_ref[...] = z - lse


ROWS_BLK = 1000


def _tc_layer(body, x, aggp, wt, b):
    return pl.pallas_call(
        body,
        out_shape=jax.ShapeDtypeStruct((N, D), jnp.float32),
        grid=(N // ROWS_BLK,),
        in_specs=[
            pl.BlockSpec((ROWS_BLK, D), lambda i: (i, 0)),
            pl.BlockSpec((NC, ROWS_BLK, D), lambda i: (0, i, 0)),
            pl.BlockSpec((D, D), lambda i: (0, 0)),
            pl.BlockSpec((1, D), lambda i: (0, 0)),
        ],
        out_specs=pl.BlockSpec((ROWS_BLK, D), lambda i: (i, 0)),
    )(x, aggp, wt, b)


def kernel(x, edge_index, W1, b1, W2, b2):
    ei = edge_index.astype(jnp.int32)
    src = ei[0].reshape(NW, NCHUNK, CHUNK)
    dst = ei[1].reshape(NW, NCHUNK, CHUNK)
    eidx = jnp.stack([src, dst], axis=2)
    zeros = jnp.zeros((RPS, D), dtype=jnp.float32)

    sc_aggregate = _make_sc_aggregate()
    agg1 = sc_aggregate(x, eidx, zeros)
    h = _tc_layer(_mm_relu_body, x, agg1, W1.T, b1.reshape(1, D))
    agg2 = sc_aggregate(h, eidx, zeros)
    out = _tc_layer(_mm_lsm_body, h, agg2, W2.T, b2.reshape(1, D))
    return out
